# Initial kernel scaffold; baseline (speedup 1.0000x reference)
import jax
import jax.numpy as jnp
from jax.experimental import pallas as pl


def kernel(x):
    raise NotImplementedError("write your pallas kernel here")


# speedup vs baseline: 17.2102x; 17.2102x over previous
import functools

import jax
import jax.numpy as jnp
from jax.experimental import pallas as pl
from jax.experimental.pallas import tpu as pltpu

_K = 64


def _topk_mask_kernel(x_ref, o_ref, *, k):
    xb = x_ref[0]
    i32 = jax.lax.bitcast_convert_type(xb, jnp.int32)
    s = jnp.where(i32 < 0, i32 ^ jnp.int32(0x7FFFFFFF), i32)

    def body(bi, cand):
        bit = jnp.left_shift(jnp.int32(1), jnp.int32(30) - bi)
        t = cand | bit
        cnt = jnp.sum((s >= t).astype(jnp.int32))
        return jnp.where(cnt >= k, t, cand)

    nonneg = jnp.sum((s >= 0).astype(jnp.int32))
    cand0 = jnp.where(nonneg >= k, jnp.int32(0), jnp.int32(-(2**31)))
    vstar = jax.lax.fori_loop(0, 31, body, cand0)
    thr_i = jnp.where(vstar < 0, vstar ^ jnp.int32(0x7FFFFFFF), vstar)
    thr = jax.lax.bitcast_convert_type(thr_i, jnp.float32)
    o_ref[0] = jnp.where(xb >= thr, xb, jnp.float32(0.0))


def kernel(x):
    b = x.shape[0]
    n = x.size // b
    assert n % 128 == 0
    rows = n // 128
    x3 = x.reshape(b, rows, 128)
    out = pl.pallas_call(
        functools.partial(_topk_mask_kernel, k=_K),
        grid=(b,),
        in_specs=[pl.BlockSpec((1, rows, 128), lambda i: (i, 0, 0))],
        out_specs=pl.BlockSpec((1, rows, 128), lambda i: (i, 0, 0)),
        out_shape=jax.ShapeDtypeStruct((b, rows, 128), jnp.float32),
        compiler_params=pltpu.CompilerParams(
            dimension_semantics=("arbitrary",),
        ),
    )(x3)
    return out.reshape(x.shape)


# speedup vs baseline: 17.2411x; 1.0018x over previous
import functools

import jax
import jax.numpy as jnp
from jax.experimental import pallas as pl
from jax.experimental.pallas import tpu as pltpu

_K = 64


def _topk_mask_kernel(x_ref, o_ref, *, k):
    xb = x_ref[0]
    i32 = jax.lax.bitcast_convert_type(xb, jnp.int32)
    s = jnp.where(i32 < 0, i32 ^ jnp.int32(0x7FFFFFFF), i32)

    nonneg = jnp.sum((s >= 0).astype(jnp.int32))
    cand = jnp.where(nonneg >= k, jnp.int32(0), jnp.int32(-(2**31)))
    for b in range(30, -1, -1):
        t = cand | jnp.int32(1 << b)
        cnt = jnp.sum((s >= t).astype(jnp.int32))
        cand = jnp.where(cnt >= k, t, cand)
    vstar = cand
    thr_i = jnp.where(vstar < 0, vstar ^ jnp.int32(0x7FFFFFFF), vstar)
    thr = jax.lax.bitcast_convert_type(thr_i, jnp.float32)
    o_ref[0] = jnp.where(xb >= thr, xb, jnp.float32(0.0))


def kernel(x):
    b = x.shape[0]
    n = x.size // b
    assert n % 128 == 0
    rows = n // 128
    x3 = x.reshape(b, rows, 128)
    out = pl.pallas_call(
        functools.partial(_topk_mask_kernel, k=_K),
        grid=(b,),
        in_specs=[pl.BlockSpec((1, rows, 128), lambda i: (i, 0, 0))],
        out_specs=pl.BlockSpec((1, rows, 128), lambda i: (i, 0, 0)),
        out_shape=jax.ShapeDtypeStruct((b, rows, 128), jnp.float32),
        compiler_params=pltpu.CompilerParams(
            dimension_semantics=("parallel",),
        ),
    )(x3)
    return out.reshape(x.shape)


# speedup vs baseline: 25.6800x; 1.4895x over previous
import functools

import jax
import jax.numpy as jnp
from jax.experimental import pallas as pl
from jax.experimental.pallas import tpu as pltpu

_K = 64


def _fold_count(m16):
    a = m16
    n = a.shape[0]
    while n > 1:
        d = next((p for p in (2, 3, 5, 7, 11, 13) if n % p == 0), n)
        step = n // d
        acc = a[0:step]
        for j in range(1, d):
            acc = acc + a[j * step:(j + 1) * step]
        a = acc
        n = step
    return jnp.sum(a[0].astype(jnp.int32))


def _bcast16(t32):
    return jnp.full((1, 16, 128), t32, jnp.int32).astype(jnp.int16)


def _count_ge(v16, t32):
    m16 = jnp.where(v16 >= _bcast16(t32), jnp.int16(1), jnp.int16(0))
    return _fold_count(m16)


def _search16(v16, cand0_if_neg, k, nbits=15):
    nonneg = _count_ge(v16, jnp.int32(0))
    cand = jnp.where(nonneg >= k, jnp.int32(0), cand0_if_neg)
    for b in range(nbits - 1, -1, -1):
        t = cand | jnp.int32(1 << b)
        cnt = _count_ge(v16, t)
        cand = jnp.where(cnt >= k, t, cand)
    return cand


def _topk_mask_kernel(x_ref, o_ref, *, k):
    xb = x_ref[0]
    i32 = jax.lax.bitcast_convert_type(xb, jnp.int32)
    s = jnp.where(i32 < 0, i32 ^ jnp.int32(0x7FFFFFFF), i32)

    tiles = s.shape[0] // 16

    s_hi = (s >> 16).astype(jnp.int16).reshape(tiles, 16, 128)
    h = _search16(s_hi, jnp.int32(-32768), k)

    hv = _bcast16(h)
    m_hi = jnp.where(s_hi > hv, jnp.int16(1), jnp.int16(0))
    cnt_gt = _fold_count(m_hi)
    kp = k - cnt_gt

    z_all = ((s & jnp.int32(0xFFFF)) - jnp.int32(32768)).astype(jnp.int16)
    z = jnp.where(s_hi == hv, z_all.reshape(tiles, 16, 128),
                  jnp.int16(-32768))
    zstar = _search16(z, jnp.int32(-32768), kp)

    lo = zstar + jnp.int32(32768)
    vstar = (h << 16) | lo
    thr_i = jnp.where(vstar < 0, vstar ^ jnp.int32(0x7FFFFFFF), vstar)
    thr = jax.lax.bitcast_convert_type(thr_i, jnp.float32)
    o_ref[0] = jnp.where(xb >= thr, xb, jnp.float32(0.0))


def kernel(x):
    b = x.shape[0]
    n = x.size // b
    assert n % 128 == 0
    rows = n // 128
    x3 = x.reshape(b, rows, 128)
    out = pl.pallas_call(
        functools.partial(_topk_mask_kernel, k=_K),
        grid=(b,),
        in_specs=[pl.BlockSpec((1, rows, 128), lambda i: (i, 0, 0))],
        out_specs=pl.BlockSpec((1, rows, 128), lambda i: (i, 0, 0)),
        out_shape=jax.ShapeDtypeStruct((b, rows, 128), jnp.float32),
        compiler_params=pltpu.CompilerParams(
            dimension_semantics=("parallel",),
        ),
    )(x3)
    return out.reshape(x.shape)


# speedup vs baseline: 37.9890x; 1.4793x over previous
import functools

import jax
import jax.numpy as jnp
from jax.experimental import pallas as pl
from jax.experimental.pallas import tpu as pltpu

_K = 64


def _fold_count(m16):
    a = m16
    n = a.shape[0]
    while n > 1:
        d = next((p for p in (2, 3, 5, 7, 11, 13) if n % p == 0), n)
        step = n // d
        acc = a[0:step]
        for j in range(1, d):
            acc = acc + a[j * step:(j + 1) * step]
        a = acc
        n = step
    return jnp.sum(a[0].astype(jnp.int32))


def _bcast16(t32, lanes):
    return jnp.full((1, 16, lanes), t32, jnp.int32).astype(jnp.int16)


def _count_ge(v16, t32):
    m16 = jnp.where(v16 >= _bcast16(t32, v16.shape[-1]),
                    jnp.int16(1), jnp.int16(0))
    return _fold_count(m16)


def _search16(v16, cand0_if_neg, k, nbits=15):
    nonneg = _count_ge(v16, jnp.int32(0))
    cand = jnp.where(nonneg >= k, jnp.int32(0), cand0_if_neg)
    for b in range(nbits - 1, -1, -1):
        t = cand | jnp.int32(1 << b)
        cnt = _count_ge(v16, t)
        cand = jnp.where(cnt >= k, t, cand)
    return cand


def _topk_mask_kernel(x_ref, o_ref, *, k, fold):
    xb = x_ref[0]
    rows, w = xb.shape
    half = rows // fold
    xp = jnp.concatenate([xb[j * half:(j + 1) * half] for j in range(fold)],
                         axis=1)
    lanes = fold * w
    tiles = half // 16

    i32 = jax.lax.bitcast_convert_type(xp, jnp.int32)
    s = jnp.where(i32 < 0, i32 ^ jnp.int32(0x7FFFFFFF), i32)

    s_hi = (s >> 16).astype(jnp.int16).reshape(tiles, 16, lanes)
    h = _search16(s_hi, jnp.int32(-32768), k)

    hv = _bcast16(h, lanes)
    m_hi = jnp.where(s_hi > hv, jnp.int16(1), jnp.int16(0))
    cnt_gt = _fold_count(m_hi)
    kp = k - cnt_gt

    z_all = ((s & jnp.int32(0xFFFF)) - jnp.int32(32768)).astype(jnp.int16)
    z = jnp.where(s_hi == hv, z_all.reshape(tiles, 16, lanes),
                  jnp.int16(-32768))
    zstar = _search16(z, jnp.int32(-32768), kp)

    lo = zstar + jnp.int32(32768)
    vstar = (h << 16) | lo
    thr_i = jnp.where(vstar < 0, vstar ^ jnp.int32(0x7FFFFFFF), vstar)
    thr = jax.lax.bitcast_convert_type(thr_i, jnp.float32)
    o_ref[0] = jnp.where(xb >= thr, xb, jnp.float32(0.0))


def kernel(x):
    b = x.shape[0]
    w = x.shape[-1]
    n = x.size // b
    rows = n // w
    fold = max(1, 128 // w)
    while fold > 1 and ((rows % fold) or ((rows // fold) % 16)):
        fold -= 1
    assert rows % 8 == 0 and (rows // fold) % 16 == 0
    x3 = x.reshape(b, rows, w)
    out = pl.pallas_call(
        functools.partial(_topk_mask_kernel, k=_K, fold=fold),
        grid=(b,),
        in_specs=[pl.BlockSpec((1, rows, w), lambda i: (i, 0, 0))],
        out_specs=pl.BlockSpec((1, rows, w), lambda i: (i, 0, 0)),
        out_shape=jax.ShapeDtypeStruct((b, rows, w), jnp.float32),
        compiler_params=pltpu.CompilerParams(
            dimension_semantics=("parallel",),
        ),
    )(x3)
    return out.reshape(x.shape)
